# baseline (speedup 1.0000x reference)
import functools
import jax
import jax.numpy as jnp
import numpy as np
from jax.experimental import pallas as pl
from jax.experimental.pallas import tpu as pltpu
from jax.experimental.pallas import tpu_sc as plsc

N_HEADS = 16
TOPK = 8
C = N_HEADS * TOPK
HP = jax.lax.Precision.HIGHEST


def _sim_kernel(te_ref, ie_ref, tW_ref, tb_ref, iW_ref, ib_ref,
                gg_ref, gb_ref, mk_ref, sim_ref):
    g = jax.lax.dot_general(te_ref[:], tW_ref[:], (((1,), (1,)), ((), ())),
                            precision=HP, preferred_element_type=jnp.float32)
    g = g + jax.lax.dot_general(ie_ref[:], iW_ref[:], (((1,), (1,)), ((), ())),
                                precision=HP, preferred_element_type=jnp.float32)
    g = g + tb_ref[:] + ib_ref[:]
    mu = jnp.mean(g, axis=1, keepdims=True)
    var = jnp.mean(jnp.square(g - mu), axis=1, keepdims=True)
    guide = (g - mu) * jax.lax.rsqrt(var + 1e-5) * gg_ref[:] + gb_ref[:]
    gnorm = jnp.sqrt(jnp.sum(guide * guide, axis=1, keepdims=True))
    gn = guide / jnp.maximum(gnorm, 1e-8)
    mk = mk_ref[:]
    knorm = jnp.sqrt(jnp.sum(mk * mk, axis=1, keepdims=True))
    kn = mk / jnp.maximum(knorm, 1e-8)
    sim_ref[:] = jax.lax.dot_general(gn, kn, (((1,), (1,)), ((), ())),
                                     precision=HP,
                                     preferred_element_type=jnp.float32)


def _make_sc_retrieve(B, M, D):
    nchunk = M // 16
    mesh = plsc.VectorSubcoreMesh(core_axis_name="c", subcore_axis_name="s")

    @functools.partial(
        pl.kernel, mesh=mesh,
        out_type=jax.ShapeDtypeStruct((B * TOPK, D), jnp.float32),
        scratch_types=[
            pltpu.VMEM((M,), jnp.float32),
            pltpu.VMEM((16,), jnp.int32),
            pltpu.VMEM((16, D), jnp.float32),
            pltpu.SemaphoreType.DMA,
        ],
    )
    def sc_retrieve(sim_hbm, mv_hbm, out_hbm, sim_v, idx_v, rows_v, sem):
        cid = jax.lax.axis_index("c")
        sid = jax.lax.axis_index("s")

        @pl.when((sid == 0) & (cid < B))
        def _():
            pltpu.sync_copy(sim_hbm.at[cid], sim_v)
            chunks = [sim_v[pl.ds(j * 16, 16)] for j in range(nchunk)]
            iota = jax.lax.iota(jnp.int32, 16)
            idxvec = jnp.zeros((16,), jnp.int32)
            neg = jnp.float32(-jnp.inf)
            for t in range(TOPK):
                vm = chunks[0]
                for j in range(1, nchunk):
                    vm = jnp.maximum(vm, chunks[j])
                for sh in (1, 2, 4, 8):
                    vm = jnp.maximum(vm, vm[iota ^ sh])
                cand = jnp.full((16,), jnp.int32(M), jnp.int32)
                for j in range(nchunk):
                    cj = jnp.where(chunks[j] == vm, iota + j * 16,
                                   jnp.int32(M))
                    cand = jnp.minimum(cand, cj)
                for sh in (1, 2, 4, 8):
                    cand = jnp.minimum(cand, cand[iota ^ sh])
                idxvec = jnp.where(iota == t, cand, idxvec)
                for j in range(nchunk):
                    chunks[j] = jnp.where(iota + j * 16 == cand, neg,
                                          chunks[j])
            idx_v[...] = idxvec
            pltpu.async_copy(mv_hbm.at[idx_v], rows_v, sem).wait()
            pltpu.sync_copy(rows_v.at[pl.ds(0, TOPK)],
                            out_hbm.at[pl.ds(cid * TOPK, TOPK)])

    return sc_retrieve


def _main_kernel(mt_ref, kb_ref, vb_ref, qb_ref, kW_ref, vW_ref, qW_ref,
                 oW_ref, x_ref, ob_ref, ng_ref, nb_ref,
                 out_ref, w1_s, b1_s, w2_s, kW_v, vW_v, qW_v, oW_v,
                 sem_k, sem_v, sem_q, sem_o):
    b = pl.program_id(0)
    s = pl.program_id(1)
    D = qW_v.shape[0]
    dh = D // N_HEADS
    B = w1_s.shape[0]

    @pl.when((b == 0) & (s == 0))
    def _fold():
        cp_k = pltpu.make_async_copy(kW_ref, kW_v, sem_k)
        cp_v = pltpu.make_async_copy(vW_ref, vW_v, sem_v)
        cp_q = pltpu.make_async_copy(qW_ref, qW_v, sem_q)
        cp_o = pltpu.make_async_copy(oW_ref, oW_v, sem_o)
        cp_k.start()
        cp_v.start()
        cp_q.start()
        cp_o.start()
        mt = mt_ref[:]
        cp_k.wait()
        K = jax.lax.dot_general(mt, kW_v[:], (((1,), (1,)), ((), ())),
                                preferred_element_type=jnp.float32) + kb_ref[:]
        cp_v.wait()
        V = jax.lax.dot_general(mt, vW_v[:], (((1,), (1,)), ((), ())),
                                preferred_element_type=jnp.float32) + vb_ref[:]
        cp_q.wait()
        cp_o.wait()
        scale = 1.0 / np.sqrt(dh)
        hc = jax.lax.broadcasted_iota(jnp.int32, (C, D), 0) // TOPK
        hd = jax.lax.broadcasted_iota(jnp.int32, (C, D), 1) // dh
        Mmask = (hc == hd).astype(jnp.float32)
        ci = jax.lax.broadcasted_iota(jnp.int32, (C, TOPK * B), 0)
        rj = jax.lax.broadcasted_iota(jnp.int32, (C, TOPK * B), 1)
        for bb in range(B):
            P = (rj == bb * TOPK + ci % TOPK).astype(jnp.float32)
            KbM = jnp.dot(P, K, preferred_element_type=jnp.float32) * Mmask
            VbM = jnp.dot(P, V, preferred_element_type=jnp.float32) * Mmask
            w1_s[bb] = scale * jax.lax.dot_general(
                qW_v[:], KbM, (((0,), (1,)), ((), ())),
                preferred_element_type=jnp.float32)
            b1_s[bb] = scale * jax.lax.dot_general(
                qb_ref[:], KbM, (((1,), (1,)), ((), ())),
                preferred_element_type=jnp.float32)
            w2_s[bb] = jax.lax.dot_general(
                VbM, oW_v[:], (((1,), (1,)), ((), ())),
                preferred_element_type=jnp.float32)

    xt = x_ref[0]
    l = (jnp.dot(xt, w1_s[b], preferred_element_type=jnp.float32) + b1_s[b])
    mx = jnp.max(l, axis=1, keepdims=True)
    e = jnp.exp(l - mx)
    gi = jax.lax.broadcasted_iota(jnp.int32, (C, C), 0) // TOPK
    gj = jax.lax.broadcasted_iota(jnp.int32, (C, C), 1) // TOPK
    G = (gi == gj).astype(jnp.float32)
    sums = jnp.dot(e, G, preferred_element_type=jnp.float32)
    p = e / sums
    y = jnp.dot(p, w2_s[b], preferred_element_type=jnp.float32) + ob_ref[:]
    r = xt + y
    mu = jnp.mean(r, axis=1, keepdims=True)
    var = jnp.mean(jnp.square(r - mu), axis=1, keepdims=True)
    out_ref[0] = (r - mu) * jax.lax.rsqrt(var + 1e-5) * ng_ref[:] + nb_ref[:]


def kernel(x, mem_keys, mem_values, text_emb, image_emb, text_W, text_b,
           img_W, img_b, gn_g, gn_b, qW, qb, kW, kb, vW, vb, oW, ob, n_g, n_b):
    B, S, D = x.shape
    M = mem_keys.shape[0]
    tb = text_b.reshape(1, -1)
    ib = img_b.reshape(1, -1)
    gg = gn_g.reshape(1, -1)
    gb = gn_b.reshape(1, -1)
    qb2 = qb.reshape(1, -1)
    kb2 = kb.reshape(1, -1)
    vb2 = vb.reshape(1, -1)
    ob2 = ob.reshape(1, -1)
    ng2 = n_g.reshape(1, -1)
    nb2 = n_b.reshape(1, -1)

    sim = pl.pallas_call(
        _sim_kernel,
        out_shape=jax.ShapeDtypeStruct((B, M), jnp.float32),
    )(text_emb, image_emb, text_W, tb, img_W, ib, gg, gb, mem_keys)

    mt = _make_sc_retrieve(B, M, D)(sim, mem_values)

    TS = 512
    full = lambda *shape: pl.BlockSpec(shape, lambda b, s: (0,) * len(shape))
    hbm = lambda: pl.BlockSpec(memory_space=pltpu.MemorySpace.HBM)
    out = pl.pallas_call(
        _main_kernel,
        grid=(B, S // TS),
        in_specs=[
            full(B * TOPK, D), full(1, D), full(1, D), full(1, D),
            hbm(), hbm(), hbm(), hbm(),
            pl.BlockSpec((1, TS, D), lambda b, s: (b, s, 0)),
            full(1, D), full(1, D), full(1, D),
        ],
        out_specs=pl.BlockSpec((1, TS, D), lambda b, s: (b, s, 0)),
        out_shape=jax.ShapeDtypeStruct((B, S, D), jnp.float32),
        scratch_shapes=[
            pltpu.VMEM((B, D, C), jnp.float32),
            pltpu.VMEM((B, 1, C), jnp.float32),
            pltpu.VMEM((B, C, D), jnp.float32),
            pltpu.VMEM((D, D), jnp.float32),
            pltpu.VMEM((D, D), jnp.float32),
            pltpu.VMEM((D, D), jnp.float32),
            pltpu.VMEM((D, D), jnp.float32),
            pltpu.SemaphoreType.DMA,
            pltpu.SemaphoreType.DMA,
            pltpu.SemaphoreType.DMA,
            pltpu.SemaphoreType.DMA,
        ],
    )(mt, kb2, vb2, qb2, kW, vW, qW, oW, x, ob2, ng2, nb2)
    return out
